# Initial kernel scaffold; baseline (speedup 1.0000x reference)
import jax
import jax.numpy as jnp
from jax.experimental import pallas as pl


def kernel(inputs, ws0, wn0, b0, ws1, wn1, b1, ws2, wn2, b2, ws3, wn3, b3, ws4, wn4, b4, ws5, wn5, b5, ws6, wn6, b6, ws7, wn7, b7, wt, bt):
    raise NotImplementedError("write your pallas kernel here")


# speedup vs baseline: 79.0078x; 79.0078x over previous
import functools

import jax
import jax.numpy as jnp
from jax.experimental import pallas as pl


def _navg(z2, H, W):
    C = z2.shape[-1]
    x3 = z2.reshape(H, W, C)
    up = jnp.concatenate([x3[1:], x3[:1]], axis=0)
    dn = jnp.concatenate([x3[-1:], x3[:-1]], axis=0)
    lf = jnp.concatenate([x3[:, 1:], x3[:, :1]], axis=1)
    rt = jnp.concatenate([x3[:, -1:], x3[:, :-1]], axis=1)
    return ((up + dn) + (lf + rt)).reshape(H * W, C) * 0.25


def _sage_pre(x2, wm, b, H, W):
    xa = jnp.concatenate([x2, _navg(x2, H, W)], axis=1)
    return jnp.dot(xa, wm, preferred_element_type=jnp.float32) + b


def _sage_post(x2, wm, b, H, W, co):
    yz = jnp.dot(x2, wm, preferred_element_type=jnp.float32)
    return yz[:, :co] + _navg(yz[:, co:], H, W) + b


def _down_body(x_ref, w0, b0, w1, b1, w2, b2, bp_ref, m_ref, *, H, W):
    x = x_ref[0]
    x0 = jax.nn.relu(_sage_pre(x, w0[...], b0[...], H, W))
    h1 = jax.nn.relu(_sage_pre(x0, w1[...], b1[...], H, W))
    h2 = jax.nn.relu(_sage_pre(h1, w2[...], b2[...], H, W))
    bp_ref[0] = h2
    x3 = h2.reshape(H, W, h2.shape[-1])
    mw = jnp.maximum(x3, jnp.concatenate([x3[:, 1:], x3[:, :1]], axis=1))
    mh = jnp.maximum(mw, jnp.concatenate([mw[1:], mw[:1]], axis=0))
    m_ref[0] = mh.reshape(H * W, h2.shape[-1])


def _low_body(p_ref, w3, b3, w4, b4, wt_ref, bt_ref, u_ref, *, H, W):
    p = p_ref[0]
    l1 = jax.nn.relu(_sage_pre(p, w3[...], b3[...], H, W))
    l2 = jax.nn.relu(_sage_pre(l1, w4[...], b4[...], H, W))
    u_ref[0] = jnp.dot(l2, wt_ref[...], preferred_element_type=jnp.float32) + bt_ref[...]


def _up_body(bp_ref, u_ref, w5, b5, w6, b6, w7, b7, o_ref, *, H, W):
    cat = jnp.concatenate([bp_ref[0], u_ref[0]], axis=1)
    c5 = jax.nn.relu(_sage_post(cat, w5[...], b5[...], H, W, w5.shape[-1] // 2))
    c6 = jax.nn.relu(_sage_pre(c5, w6[...], b6[...], H, W))
    o_ref[0] = _sage_pre(c6, w7[...], b7[...], H, W)


def _const_spec(shape):
    nd = len(shape)
    return pl.BlockSpec(shape, lambda i: (0,) * nd)


def _tile_spec(shape):
    return pl.BlockSpec((1,) + shape[1:], lambda i: (i,) + (0,) * (len(shape) - 1))


def kernel(inputs, ws0, wn0, b0, ws1, wn1, b1, ws2, wn2, b2, ws3, wn3, b3,
           ws4, wn4, b4, ws5, wn5, b5, ws6, wn6, b6, ws7, wn7, b7, wt, bt):
    B, T, NX, NY, C = inputs.shape
    BT = B * T
    HW = NX * NY
    H2, W2 = NX // 2, NY // 2
    f32 = jnp.float32

    w0 = jnp.concatenate([ws0, wn0], axis=0)
    w1 = jnp.concatenate([ws1, wn1], axis=0)
    w2 = jnp.concatenate([ws2, wn2], axis=0)
    w3 = jnp.concatenate([ws3, wn3], axis=0)
    w4 = jnp.concatenate([ws4, wn4], axis=0)
    w5 = jnp.concatenate([ws5, wn5], axis=1)
    w6 = jnp.concatenate([ws6, wn6], axis=0)
    w7 = jnp.concatenate([ws7, wn7], axis=0)
    wt2 = wt.transpose(0, 2, 3, 1).reshape(wt.shape[0], -1)
    bt4 = jnp.tile(bt, 4).reshape(1, -1)
    b0r, b1r, b2r, b3r, b4r, b5r, b6r, b7r = (
        b.reshape(1, -1) for b in (b0, b1, b2, b3, b4, b5, b6, b7))

    x = inputs.reshape(BT, HW, C)

    bp, m2d = pl.pallas_call(
        functools.partial(_down_body, H=NX, W=NY),
        grid=(BT,),
        in_specs=[_tile_spec(x.shape)] + [
            _const_spec(a.shape) for a in (w0, b0r, w1, b1r, w2, b2r)],
        out_specs=[_tile_spec((BT, HW, 256)), _tile_spec((BT, HW, 256))],
        out_shape=[jax.ShapeDtypeStruct((BT, HW, 256), f32),
                   jax.ShapeDtypeStruct((BT, HW, 256), f32)],
    )(x, w0, b0r, w1, b1r, w2, b2r)

    p = m2d.reshape(B, T, NX, NY, 256)[:, :, ::2, ::2, :].reshape(BT, HW // 4, 256)

    u = pl.pallas_call(
        functools.partial(_low_body, H=H2, W=W2),
        grid=(BT,),
        in_specs=[_tile_spec(p.shape)] + [
            _const_spec(a.shape) for a in (w3, b3r, w4, b4r, wt2, bt4)],
        out_specs=[_tile_spec((BT, HW // 4, wt2.shape[1]))],
        out_shape=[jax.ShapeDtypeStruct((BT, HW // 4, wt2.shape[1]), f32)],
    )(p, w3, b3r, w4, b4r, wt2, bt4)[0]

    uco = wt.shape[1]
    u48 = (u.reshape(B, T, H2, W2, 2, 2, uco)
           .transpose(0, 1, 2, 4, 3, 5, 6)
           .reshape(BT, HW, uco))

    out = pl.pallas_call(
        functools.partial(_up_body, H=NX, W=NY),
        grid=(BT,),
        in_specs=[_tile_spec(bp.shape), _tile_spec(u48.shape)] + [
            _const_spec(a.shape) for a in (w5, b5r, w6, b6r, w7, b7r)],
        out_specs=[_tile_spec((BT, HW, w7.shape[1]))],
        out_shape=[jax.ShapeDtypeStruct((BT, HW, w7.shape[1]), f32)],
    )(bp, u48, w5, b5r, w6, b6r, w7, b7r)[0]

    return out.reshape(B, T, NX, NY, w7.shape[1])


# speedup vs baseline: 132.8229x; 1.6811x over previous
import functools

import jax
import jax.numpy as jnp
from jax.experimental import pallas as pl
from jax.experimental.pallas import tpu as pltpu

_PARAMS = pltpu.CompilerParams(dimension_semantics=("parallel",))


def _navg(z2, H, W):
    C = z2.shape[-1]
    x3 = z2.reshape(H, W, C)
    up = jnp.concatenate([x3[1:], x3[:1]], axis=0)
    dn = jnp.concatenate([x3[-1:], x3[:-1]], axis=0)
    lf = jnp.concatenate([x3[:, 1:], x3[:, :1]], axis=1)
    rt = jnp.concatenate([x3[:, -1:], x3[:, :-1]], axis=1)
    return ((up + dn) + (lf + rt)).reshape(H * W, C) * 0.25


def _sage_pre(x2, wm, b, H, W):
    xa = jnp.concatenate([x2, _navg(x2, H, W)], axis=1)
    return jnp.dot(xa, wm, preferred_element_type=jnp.float32) + b


def _sage_post(x2, wm, b, H, W, co):
    yz = jnp.dot(x2, wm, preferred_element_type=jnp.float32)
    return yz[:, :co] + _navg(yz[:, co:], H, W) + b


def _down_body(x_ref, w0, b0, w1, b1, w2, b2, bp_ref, m_ref, *, H, W):
    x = x_ref[0]
    x0 = jax.nn.relu(_sage_pre(x, w0[...], b0[...], H, W))
    h1 = jax.nn.relu(_sage_pre(x0, w1[...], b1[...], H, W))
    h2 = jax.nn.relu(_sage_pre(h1, w2[...], b2[...], H, W))
    bp_ref[0] = h2
    C = h2.shape[-1]
    x3 = h2.reshape(H, W, C)
    mw = jnp.maximum(x3, jnp.concatenate([x3[:, 1:], x3[:, :1]], axis=1))
    mh = jnp.maximum(mw, jnp.concatenate([mw[1:], mw[:1]], axis=0))
    pe = mh.reshape(H // 2, 2, W, C)[:, 0]
    m_ref[0] = pe.reshape(H * W // 2, C)


def _low_body(p_ref, w3, b3, w4, b4, wt_ref, bt_ref, u_ref, *, H, W):
    p = p_ref[0]
    l1 = jax.nn.relu(_sage_pre(p, w3[...], b3[...], H, W))
    l2 = jax.nn.relu(_sage_pre(l1, w4[...], b4[...], H, W))
    u_ref[0] = jnp.dot(l2, wt_ref[...], preferred_element_type=jnp.float32) + bt_ref[...]


def _up_body(bp_ref, u_ref, w5, b5, w6, b6, w7, b7, o_ref, *, H, W):
    cat = jnp.concatenate([bp_ref[0], u_ref[0]], axis=1)
    c5 = jax.nn.relu(_sage_post(cat, w5[...], b5[...], H, W, w5.shape[-1] // 2))
    c6 = jax.nn.relu(_sage_pre(c5, w6[...], b6[...], H, W))
    o_ref[0] = _sage_pre(c6, w7[...], b7[...], H, W)


def _const_spec(shape):
    nd = len(shape)
    return pl.BlockSpec(shape, lambda i: (0,) * nd)


def _tile_spec(shape):
    return pl.BlockSpec((1,) + shape[1:], lambda i: (i,) + (0,) * (len(shape) - 1))


def kernel(inputs, ws0, wn0, b0, ws1, wn1, b1, ws2, wn2, b2, ws3, wn3, b3,
           ws4, wn4, b4, ws5, wn5, b5, ws6, wn6, b6, ws7, wn7, b7, wt, bt):
    B, T, NX, NY, C = inputs.shape
    BT = B * T
    HW = NX * NY
    H2, W2 = NX // 2, NY // 2
    f32 = jnp.float32

    w0 = jnp.concatenate([ws0, wn0], axis=0)
    w1 = jnp.concatenate([ws1, wn1], axis=0)
    w2 = jnp.concatenate([ws2, wn2], axis=0)
    w3 = jnp.concatenate([ws3, wn3], axis=0)
    w4 = jnp.concatenate([ws4, wn4], axis=0)
    w5 = jnp.concatenate([ws5, wn5], axis=1)
    w6 = jnp.concatenate([ws6, wn6], axis=0)
    w7 = jnp.concatenate([ws7, wn7], axis=0)
    wt2 = wt.transpose(0, 2, 3, 1).reshape(wt.shape[0], -1)
    bt4 = jnp.tile(bt, 4).reshape(1, -1)
    b0r, b1r, b2r, b3r, b4r, b5r, b6r, b7r = (
        b.reshape(1, -1) for b in (b0, b1, b2, b3, b4, b5, b6, b7))

    x = inputs.reshape(BT, HW, C)

    bp, m2d = pl.pallas_call(
        functools.partial(_down_body, H=NX, W=NY),
        grid=(BT,),
        in_specs=[_tile_spec(x.shape)] + [
            _const_spec(a.shape) for a in (w0, b0r, w1, b1r, w2, b2r)],
        out_specs=[_tile_spec((BT, HW, 256)), _tile_spec((BT, HW // 2, 256))],
        out_shape=[jax.ShapeDtypeStruct((BT, HW, 256), f32),
                   jax.ShapeDtypeStruct((BT, HW // 2, 256), f32)],
        compiler_params=_PARAMS,
    )(x, w0, b0r, w1, b1r, w2, b2r)

    p = m2d.reshape(B, T, H2, NY, 256)[:, :, :, ::2, :].reshape(BT, HW // 4, 256)

    u = pl.pallas_call(
        functools.partial(_low_body, H=H2, W=W2),
        grid=(BT,),
        in_specs=[_tile_spec(p.shape)] + [
            _const_spec(a.shape) for a in (w3, b3r, w4, b4r, wt2, bt4)],
        out_specs=[_tile_spec((BT, HW // 4, wt2.shape[1]))],
        out_shape=[jax.ShapeDtypeStruct((BT, HW // 4, wt2.shape[1]), f32)],
        compiler_params=_PARAMS,
    )(p, w3, b3r, w4, b4r, wt2, bt4)[0]

    uco = wt.shape[1]
    u48 = (u.reshape(B, T, H2, W2, 2, 2, uco)
           .transpose(0, 1, 2, 4, 3, 5, 6)
           .reshape(BT, HW, uco))

    out = pl.pallas_call(
        functools.partial(_up_body, H=NX, W=NY),
        grid=(BT,),
        in_specs=[_tile_spec(bp.shape), _tile_spec(u48.shape)] + [
            _const_spec(a.shape) for a in (w5, b5r, w6, b6r, w7, b7r)],
        out_specs=[_tile_spec((BT, HW, w7.shape[1]))],
        out_shape=[jax.ShapeDtypeStruct((BT, HW, w7.shape[1]), f32)],
        compiler_params=_PARAMS,
    )(bp, u48, w5, b5r, w6, b6r, w7, b7r)[0]

    return out.reshape(B, T, NX, NY, w7.shape[1])


# speedup vs baseline: 260.7648x; 1.9633x over previous
import functools

import jax
import jax.numpy as jnp
from jax.experimental import pallas as pl
from jax.experimental.pallas import tpu as pltpu

_PARAMS = pltpu.CompilerParams(dimension_semantics=("parallel",))


def _navg(z2, H, W):
    C = z2.shape[-1]
    x3 = z2.reshape(H, W, C)
    up = jnp.concatenate([x3[1:], x3[:1]], axis=0)
    dn = jnp.concatenate([x3[-1:], x3[:-1]], axis=0)
    lf = jnp.concatenate([x3[:, 1:], x3[:, :1]], axis=1)
    rt = jnp.concatenate([x3[:, -1:], x3[:, :-1]], axis=1)
    return ((up + dn) + (lf + rt)).reshape(H * W, C) * 0.25


def _sage_pre(x2, wm, b, H, W):
    xa = jnp.concatenate([x2, _navg(x2, H, W)], axis=1)
    return jnp.dot(xa, wm, preferred_element_type=jnp.float32) + b


def _sage_post(x2, wm, b, H, W, co):
    yz = jnp.dot(x2, wm, preferred_element_type=jnp.float32)
    return yz[:, :co] + _navg(yz[:, co:], H, W) + b


def _body(x_ref, w0, b0, w1, b1, w2, b2, w3, b3, w4, b4, wt, bt,
          w5, b5, w6, b6, w7, b7, o_ref, *, H, W):
    relu = jax.nn.relu
    x = x_ref[0]
    x0 = relu(_sage_pre(x, w0[...], b0[...], H, W))
    h1 = relu(_sage_pre(x0, w1[...], b1[...], H, W))
    h2 = relu(_sage_pre(h1, w2[...], b2[...], H, W))

    C = h2.shape[-1]
    x3 = h2.reshape(H, W, C)
    mw = jnp.maximum(x3, jnp.concatenate([x3[:, 1:], x3[:, :1]], axis=1))
    mh = jnp.maximum(mw, jnp.concatenate([mw[1:], mw[:1]], axis=0))
    pe = mh.reshape(H // 2, 2, W, C)[:, 0]
    p = pltpu.einshape("a(bp)c->pabc", pe, p=2)[0]
    p = p.reshape(H * W // 4, C)

    h2d, w2d = H // 2, W // 2
    l1 = relu(_sage_pre(p, w3[...], b3[...], h2d, w2d))
    l2 = relu(_sage_pre(l1, w4[...], b4[...], h2d, w2d))

    u = jnp.dot(l2, wt[...], preferred_element_type=jnp.float32) + bt[...]
    uc = u.shape[-1] // 4
    u5 = u.reshape(h2d, w2d, 2, 2, uc)
    u48 = pltpu.einshape("hwklc->(hk)(wl)c", u5).reshape(H * W, uc)

    cat = jnp.concatenate([h2, u48], axis=1)
    c5 = relu(_sage_post(cat, w5[...], b5[...], H, W, w5.shape[-1] // 2))
    c6 = relu(_sage_pre(c5, w6[...], b6[...], H, W))
    o_ref[0] = _sage_pre(c6, w7[...], b7[...], H, W)


def _const_spec(shape):
    nd = len(shape)
    return pl.BlockSpec(shape, lambda i: (0,) * nd)


def _tile_spec(shape):
    return pl.BlockSpec((1,) + shape[1:], lambda i: (i,) + (0,) * (len(shape) - 1))


def kernel(inputs, ws0, wn0, b0, ws1, wn1, b1, ws2, wn2, b2, ws3, wn3, b3,
           ws4, wn4, b4, ws5, wn5, b5, ws6, wn6, b6, ws7, wn7, b7, wt, bt):
    B, T, NX, NY, C = inputs.shape
    BT = B * T
    HW = NX * NY
    f32 = jnp.float32

    w0 = jnp.concatenate([ws0, wn0], axis=0)
    w1 = jnp.concatenate([ws1, wn1], axis=0)
    w2 = jnp.concatenate([ws2, wn2], axis=0)
    w3 = jnp.concatenate([ws3, wn3], axis=0)
    w4 = jnp.concatenate([ws4, wn4], axis=0)
    w5 = jnp.concatenate([ws5, wn5], axis=1)
    w6 = jnp.concatenate([ws6, wn6], axis=0)
    w7 = jnp.concatenate([ws7, wn7], axis=0)
    wt2 = wt.transpose(0, 2, 3, 1).reshape(wt.shape[0], -1)
    bt4 = jnp.tile(bt, 4).reshape(1, -1)
    b0r, b1r, b2r, b3r, b4r, b5r, b6r, b7r = (
        b.reshape(1, -1) for b in (b0, b1, b2, b3, b4, b5, b6, b7))

    x = inputs.reshape(BT, HW, C)
    consts = (w0, b0r, w1, b1r, w2, b2r, w3, b3r, w4, b4r, wt2, bt4,
              w5, b5r, w6, b6r, w7, b7r)

    out = pl.pallas_call(
        functools.partial(_body, H=NX, W=NY),
        grid=(BT,),
        in_specs=[_tile_spec(x.shape)] + [_const_spec(a.shape) for a in consts],
        out_specs=[_tile_spec((BT, HW, w7.shape[1]))],
        out_shape=[jax.ShapeDtypeStruct((BT, HW, w7.shape[1]), f32)],
        compiler_params=_PARAMS,
    )(x, *consts)[0]

    return out.reshape(B, T, NX, NY, w7.shape[1])
